# baseline (speedup 1.0000x reference)
import functools

import jax
import jax.numpy as jnp
from jax import lax
from jax.experimental import pallas as pl
from jax.experimental.pallas import tpu as pltpu
from jax.experimental.pallas import tpu_sc as plsc

NC = 2
NS = 16
NW = NC * NS
C = 128
RPT = 640
NP = NS * RPT


def _sc_mesh():
    return plsc.VectorSubcoreMesh(
        core_axis_name="c", subcore_axis_name="s", num_cores=NC, num_subcores=NS
    )


@functools.lru_cache(maxsize=None)
def _sc_aggregate_prog(ch, c, d):
    nzc = RPT // c

    @functools.partial(
        pl.kernel,
        out_type=jax.ShapeDtypeStruct((NW, RPT, d), jnp.float32),
        mesh=_sc_mesh(),
        scratch_types=[
            pltpu.VMEM((c,), jnp.int32),
            pltpu.VMEM((c,), jnp.int32),
            pltpu.VMEM((c, d), jnp.float32),
            pltpu.VMEM_SHARED((NP, d), jnp.float32),
            pltpu.SemaphoreType.DMA,
        ],
    )
    def k(h_h, src_h, dst_h, zeros_h, out_h, sidx, didx, rows_v,
          agg_sp, sem):
        core = lax.axis_index("c")
        sub = lax.axis_index("s")
        wid = core * NS + sub
        r0 = sub * RPT
        pltpu.sync_copy(zeros_h, rows_v)
        for kk in range(nzc):
            pltpu.sync_copy(rows_v, agg_sp.at[pl.ds(r0 + kk * c, c)])
        plsc.subcore_barrier()

        def body(j, carry):
            pltpu.sync_copy(src_h.at[wid, j], sidx)
            pltpu.sync_copy(dst_h.at[wid, j], didx)
            pltpu.async_copy(h_h.at[sidx], rows_v, sem).wait()
            pltpu.sync_copy(rows_v, agg_sp.at[didx], add=True)
            return carry

        lax.fori_loop(0, ch, body, 0)
        plsc.subcore_barrier()
        for kk in range(nzc):
            pltpu.sync_copy(agg_sp.at[pl.ds(r0 + kk * c, c)], rows_v)
            pltpu.sync_copy(rows_v, out_h.at[wid, pl.ds(kk * c, c)])

    return k


def _sc_aggregate(h, src3d, dst3d, zeros, d):
    _, ch, c = src3d.shape
    return _sc_aggregate_prog(ch, c, d)(h, src3d, dst3d, zeros)


def _tc_prep(x, dout_p):
    n, d = x.shape
    blk = 1000

    def body(x_ref, dg_ref, o_ref):
        deg = dg_ref[0][:, 0:1] + dg_ref[1][:, 0:1]
        norm = lax.rsqrt(jnp.maximum(deg, 1.0))
        o_ref[...] = x_ref[...] * norm

    return pl.pallas_call(
        body,
        grid=(n // blk,),
        in_specs=[
            pl.BlockSpec((blk, d), lambda i: (i, 0)),
            pl.BlockSpec((NC, blk, d), lambda i: (0, i, 0)),
        ],
        out_specs=pl.BlockSpec((blk, d), lambda i: (i, 0)),
        out_shape=jax.ShapeDtypeStruct((n, d), jnp.float32),
    )(x, dout_p)


def _tc_finish(parts, din_p, W, b2d, dout_p, n):
    d = parts.shape[2]
    h = W.shape[1]
    blk = 1000
    prep = dout_p is not None

    def body(*refs):
        if prep:
            p_ref, di_ref, W_ref, b_ref, do_ref, o_ref = refs
        else:
            p_ref, di_ref, W_ref, b_ref, o_ref = refs
        agg = p_ref[0] + p_ref[1]
        din = di_ref[0][:, 0:1] + di_ref[1][:, 0:1]
        agg = agg * lax.rsqrt(jnp.maximum(din, 1.0))
        y = jnp.dot(agg, W_ref[...], preferred_element_type=jnp.float32)
        y = jnp.maximum(y + b_ref[...], 0.0)
        if prep:
            dout = do_ref[0][:, 0:1] + do_ref[1][:, 0:1]
            y = y * lax.rsqrt(jnp.maximum(dout, 1.0))
        o_ref[...] = y

    in_specs = [
        pl.BlockSpec((NC, blk, d), lambda i: (0, i, 0)),
        pl.BlockSpec((NC, blk, d), lambda i: (0, i, 0)),
        pl.BlockSpec((d, h), lambda i: (0, 0)),
        pl.BlockSpec((1, h), lambda i: (0, 0)),
    ]
    args = [parts, din_p, W, b2d]
    if prep:
        in_specs.append(pl.BlockSpec((NC, blk, d), lambda i: (0, i, 0)))
        args.append(dout_p)
    return pl.pallas_call(
        body,
        grid=(n // blk,),
        in_specs=in_specs,
        out_specs=pl.BlockSpec((blk, h), lambda i: (i, 0)),
        out_shape=jax.ShapeDtypeStruct((n, h), jnp.float32),
    )(*args)


def kernel(inputs, edge_index, W1, b1, W2, b2):
    n, d = inputs.shape
    e = edge_index.shape[1]
    hdim = W1.shape[1]
    assert e % NW == 0 and n < NP
    ep = e // NW
    epp = -(-ep // C) * C
    ch = epp // C
    npad = epp - ep

    srcr = edge_index[0].reshape(NW, ep)
    dstr = edge_index[1].reshape(NW, ep)
    pad0 = jnp.zeros((NW, npad), jnp.int32)
    padn = jnp.full((NW, npad), n, jnp.int32)
    agg_src = jnp.concatenate([srcr, pad0], axis=1).reshape(NW, ch, C)
    deg_src = jnp.concatenate([srcr, padn], axis=1).reshape(NW, ch, C)
    dst3d = jnp.concatenate([dstr, padn], axis=1).reshape(NW, ch, C)
    zidx = jnp.zeros((NW, ch, C), jnp.int32)
    ones_mat = jnp.ones((n, d), jnp.float32)
    zeros_agg = jnp.zeros((C, d), jnp.float32)

    din_p = _sc_aggregate(ones_mat, zidx, dst3d, zeros_agg, d).reshape(NC, NP, d)
    dout_p = _sc_aggregate(ones_mat, zidx, deg_src, zeros_agg, d).reshape(NC, NP, d)

    b1_2d = b1.reshape(1, hdim)
    b2_2d = b2.reshape(1, hdim)

    h0 = _tc_prep(inputs, dout_p)
    p1 = _sc_aggregate(h0, agg_src, dst3d, zeros_agg, d).reshape(NC, NP, d)
    h1 = _tc_finish(p1, din_p, W1, b1_2d, dout_p, n)
    p2 = _sc_aggregate(h1, agg_src, dst3d, zeros_agg, hdim).reshape(NC, NP, hdim)
    h2 = _tc_finish(p2, din_p, W2, b2_2d, None, n)
    return h2


# speedup vs baseline: 16.4503x; 16.4503x over previous
import functools

import jax
import jax.numpy as jnp
from jax import lax
from jax.experimental import pallas as pl
from jax.experimental.pallas import tpu as pltpu
from jax.experimental.pallas import tpu_sc as plsc

NC = 2
NS = 16
NW = NC * NS
C = 128
RPT = 640
NP = NS * RPT


def _sc_mesh():
    return plsc.VectorSubcoreMesh(
        core_axis_name="c", subcore_axis_name="s", num_cores=NC, num_subcores=NS
    )


@functools.lru_cache(maxsize=None)
def _sc_aggregate_prog(ch, c, d):
    nzc = RPT // c

    @functools.partial(
        pl.kernel,
        out_type=jax.ShapeDtypeStruct((NW, RPT, d), jnp.float32),
        mesh=_sc_mesh(),
        scratch_types=[
            pltpu.VMEM((c,), jnp.int32),
            pltpu.VMEM((c,), jnp.int32),
            pltpu.VMEM((c, d), jnp.float32),
            pltpu.VMEM_SHARED((NP, d), jnp.float32),
            pltpu.SemaphoreType.DMA,
        ],
    )
    def k(h_h, src_h, dst_h, zeros_h, out_h, sidx, didx, rows_v,
          agg_sp, sem):
        core = lax.axis_index("c")
        sub = lax.axis_index("s")
        wid = core * NS + sub
        r0 = sub * RPT
        pltpu.sync_copy(zeros_h, rows_v)
        for kk in range(nzc):
            pltpu.sync_copy(rows_v, agg_sp.at[pl.ds(r0 + kk * c, c)])
        plsc.subcore_barrier()

        def body(j, carry):
            pltpu.sync_copy(src_h.at[wid, j], sidx)
            pltpu.sync_copy(dst_h.at[wid, j], didx)
            pltpu.async_copy(h_h.at[sidx], rows_v, sem).wait()
            pltpu.sync_copy(rows_v, agg_sp.at[didx], add=True)
            return carry

        lax.fori_loop(0, ch, body, 0)
        plsc.subcore_barrier()
        for kk in range(nzc):
            pltpu.sync_copy(agg_sp.at[pl.ds(r0 + kk * c, c)], rows_v)
            pltpu.sync_copy(rows_v, out_h.at[wid, pl.ds(kk * c, c)])

    return k


def _sc_aggregate(h, src3d, dst3d, zeros, d):
    _, ch, c = src3d.shape
    return _sc_aggregate_prog(ch, c, d)(h, src3d, dst3d, zeros)


def _tc_prep(x, dout_p):
    n, d = x.shape
    blk = 1000

    def body(x_ref, dg_ref, o_ref):
        deg = dg_ref[0][:, 0:1] + dg_ref[1][:, 0:1]
        norm = lax.rsqrt(jnp.maximum(deg, 1.0))
        o_ref[...] = x_ref[...] * norm

    return pl.pallas_call(
        body,
        grid=(n // blk,),
        in_specs=[
            pl.BlockSpec((blk, d), lambda i: (i, 0)),
            pl.BlockSpec((NC, blk, d), lambda i: (0, i, 0)),
        ],
        out_specs=pl.BlockSpec((blk, d), lambda i: (i, 0)),
        out_shape=jax.ShapeDtypeStruct((n, d), jnp.float32),
    )(x, dout_p)


def _tc_finish(parts, din_p, W, b2d, dout_p, n):
    d = parts.shape[2]
    h = W.shape[1]
    blk = 1000
    prep = dout_p is not None

    def body(*refs):
        if prep:
            p_ref, di_ref, W_ref, b_ref, do_ref, o_ref = refs
        else:
            p_ref, di_ref, W_ref, b_ref, o_ref = refs
        agg = p_ref[0] + p_ref[1]
        din = di_ref[0][:, 0:1] + di_ref[1][:, 0:1]
        agg = agg * lax.rsqrt(jnp.maximum(din, 1.0))
        y = jnp.dot(agg, W_ref[...], preferred_element_type=jnp.float32)
        y = jnp.maximum(y + b_ref[...], 0.0)
        if prep:
            dout = do_ref[0][:, 0:1] + do_ref[1][:, 0:1]
            y = y * lax.rsqrt(jnp.maximum(dout, 1.0))
        o_ref[...] = y

    in_specs = [
        pl.BlockSpec((NC, blk, d), lambda i: (0, i, 0)),
        pl.BlockSpec((NC, blk, d), lambda i: (0, i, 0)),
        pl.BlockSpec((d, h), lambda i: (0, 0)),
        pl.BlockSpec((1, h), lambda i: (0, 0)),
    ]
    args = [parts, din_p, W, b2d]
    if prep:
        in_specs.append(pl.BlockSpec((NC, blk, d), lambda i: (0, i, 0)))
        args.append(dout_p)
    return pl.pallas_call(
        body,
        grid=(n // blk,),
        in_specs=in_specs,
        out_specs=pl.BlockSpec((blk, h), lambda i: (i, 0)),
        out_shape=jax.ShapeDtypeStruct((n, h), jnp.float32),
    )(*args)


def kernel(inputs, edge_index, W1, b1, W2, b2):
    n, d = inputs.shape
    e = edge_index.shape[1]
    hdim = W1.shape[1]
    assert e % NW == 0 and n < NP
    ep = e // NW
    epp = -(-ep // C) * C
    ch = epp // C
    npad = epp - ep

    srcr = edge_index[0].reshape(NW, ep)
    dstr = edge_index[1].reshape(NW, ep)
    pad0 = jnp.zeros((NW, npad), jnp.int32)
    padn = jnp.full((NW, npad), n, jnp.int32)
    agg_src = jnp.concatenate([srcr, pad0], axis=1).reshape(NW, ch, C)
    deg_src = jnp.concatenate([srcr, padn], axis=1).reshape(NW, ch, C)
    dst3d = jnp.concatenate([dstr, padn], axis=1).reshape(NW, ch, C)
    ones_mat = jnp.ones((n, d), jnp.float32)
    zeros_agg = jnp.zeros((C, d), jnp.float32)

    din_p = _sc_aggregate(ones_mat, agg_src, dst3d, zeros_agg, d).reshape(NC, NP, d)
    dout_p = _sc_aggregate(ones_mat, agg_src, deg_src, zeros_agg, d).reshape(NC, NP, d)

    b1_2d = b1.reshape(1, hdim)
    b2_2d = b2.reshape(1, hdim)

    h0 = _tc_prep(inputs, dout_p)
    p1 = _sc_aggregate(h0, agg_src, dst3d, zeros_agg, d).reshape(NC, NP, d)
    h1 = _tc_finish(p1, din_p, W1, b1_2d, dout_p, n)
    p2 = _sc_aggregate(h1, agg_src, dst3d, zeros_agg, hdim).reshape(NC, NP, hdim)
    h2 = _tc_finish(p2, din_p, W2, b2_2d, None, n)
    return h2
